# baseline (speedup 1.0000x reference)
import functools

import jax
import jax.numpy as jnp
from jax import lax
from jax.experimental import pallas as pl
from jax.experimental.pallas import tpu as pltpu
from jax.experimental.pallas import tpu_sc as plsc

_BATCH = 16384
_DIM = 2
_NC = 2
_NS = 16
_LANES = 16
_NW = _NC * _NS
_BPW = _BATCH // _NW
_CHUNK = 128
_NCHUNK = _BPW // _CHUNK


def _sc_body(users_hbm, items_hbm, uef_hbm, ief_hbm, wb_hbm, out_hbm,
             idx_u, idx_i, eu, ou, ei, oi, u0, u1, i0, i1, out_v, w_v, sem):
    wid = lax.axis_index("s") * _NC + lax.axis_index("c")
    base = wid * _BPW

    pltpu.sync_copy(users_hbm.at[pl.ds(base, _BPW)], idx_u)
    pltpu.sync_copy(items_hbm.at[pl.ds(base, _BPW)], idx_i)
    pltpu.sync_copy(wb_hbm, w_v)

    def mkidx(k, _):
        sl = pl.ds(k * _LANES, _LANES)
        us = idx_u[sl]
        its = idx_i[sl]
        ue = us * 2
        ie = its * 2
        eu[sl] = ue
        ou[sl] = ue + 1
        ei[sl] = ie
        oi[sl] = ie + 1
        return 0

    lax.fori_loop(0, _BPW // _LANES, mkidx, 0, unroll=4)

    copies = []
    for j in range(_NCHUNK):
        sl = pl.ds(j * _CHUNK, _CHUNK)
        copies.append(pltpu.async_copy(uef_hbm.at[eu.at[sl]], u0.at[sl], sem))
        copies.append(pltpu.async_copy(uef_hbm.at[ou.at[sl]], u1.at[sl], sem))
        copies.append(pltpu.async_copy(ief_hbm.at[ei.at[sl]], i0.at[sl], sem))
        copies.append(pltpu.async_copy(ief_hbm.at[oi.at[sl]], i1.at[sl], sem))
    for c in copies:
        c.wait()

    wv = w_v[...]
    w0 = wv[0]
    w1 = wv[1]
    bias = wv[2]

    def step(k, _):
        sl = pl.ds(k * _LANES, _LANES)
        out_v[sl] = (u0[sl] + i0[sl]) * w0 + (u1[sl] + i1[sl]) * w1 + bias
        return 0

    lax.fori_loop(0, _BPW // _LANES, step, 0, unroll=4)

    pltpu.sync_copy(out_v, out_hbm.at[pl.ds(base, _BPW)])


_sc_kernel = functools.partial(
    pl.kernel,
    out_type=jax.ShapeDtypeStruct((_BATCH,), jnp.float32),
    mesh=plsc.VectorSubcoreMesh(core_axis_name="c", subcore_axis_name="s",
                                num_cores=_NC, num_subcores=_NS),
    scratch_types=[
        pltpu.VMEM((_BPW,), jnp.int32),
        pltpu.VMEM((_BPW,), jnp.int32),
        pltpu.VMEM((_BPW,), jnp.int32),
        pltpu.VMEM((_BPW,), jnp.int32),
        pltpu.VMEM((_BPW,), jnp.int32),
        pltpu.VMEM((_BPW,), jnp.int32),
        pltpu.VMEM((_BPW,), jnp.float32),
        pltpu.VMEM((_BPW,), jnp.float32),
        pltpu.VMEM((_BPW,), jnp.float32),
        pltpu.VMEM((_BPW,), jnp.float32),
        pltpu.VMEM((_BPW,), jnp.float32),
        pltpu.VMEM((_LANES,), jnp.float32),
        pltpu.SemaphoreType.DMA,
    ],
    compiler_params=pltpu.CompilerParams(use_tc_tiling_on_sc=False),
)(_sc_body)


def kernel(users, items, UE, IE, FC_W, FC_b):
    wb = jnp.concatenate(
        [FC_W.reshape(_DIM), FC_b, jnp.zeros((_LANES - _DIM - 1,), jnp.float32)])
    out = _sc_kernel(users, items, UE.reshape(-1), IE.reshape(-1), wb)
    return out.reshape(_BATCH, 1)


# speedup vs baseline: 61.5388x; 61.5388x over previous
import functools

import jax
import jax.numpy as jnp
from jax import lax
from jax.experimental import pallas as pl
from jax.experimental.layout import Layout, with_layout_constraint
from jax.experimental.pallas import tpu as pltpu
from jax.experimental.pallas import tpu_sc as plsc

_BATCH = 16384
_DIM = 2
_NROWS = 1000000
_MAIN = 999936
_NTAIL = _NROWS - _MAIN
_NC = 2
_NS = 16
_LANES = 16
_NW = _NC * _NS
_BPW = _BATCH // _NW
_CHUNK = 128
_NCHUNK = _BPW // _CHUNK


def _sc_body(users_hbm, items_hbm, uef_hbm, ief_hbm, tailu_hbm, taili_hbm,
             wb_hbm, out_hbm,
             idx_u, idx_i, fu0, fu1, fi0, fi1, u0, u1, i0, i1,
             tailu_v, taili_v, out_v, w_v, sem):
    wid = lax.axis_index("s") * _NC + lax.axis_index("c")
    base = wid * _BPW

    pltpu.sync_copy(users_hbm.at[pl.ds(base, _BPW)], idx_u)
    pltpu.sync_copy(items_hbm.at[pl.ds(base, _BPW)], idx_i)
    pltpu.sync_copy(tailu_hbm, tailu_v)
    pltpu.sync_copy(taili_hbm, taili_v)
    pltpu.sync_copy(wb_hbm, w_v)

    def mkidx(k, _):
        sl = pl.ds(k * _LANES, _LANES)
        ru = idx_u[sl]
        ri = idx_i[sl]
        rum = jnp.where(ru < _MAIN, ru, 0)
        rim = jnp.where(ri < _MAIN, ri, 0)
        bu = lax.shift_right_logical(rum, 7) * 256 + (rum & 127)
        bi = lax.shift_right_logical(rim, 7) * 256 + (rim & 127)
        fu0[sl] = bu
        fu1[sl] = bu + 128
        fi0[sl] = bi
        fi1[sl] = bi + 128
        return 0

    lax.fori_loop(0, _BPW // _LANES, mkidx, 0, unroll=4)

    copies = []
    for j in range(_NCHUNK):
        sl = pl.ds(j * _CHUNK, _CHUNK)
        copies.append(pltpu.async_copy(uef_hbm.at[fu0.at[sl]], u0.at[sl], sem))
        copies.append(pltpu.async_copy(uef_hbm.at[fu1.at[sl]], u1.at[sl], sem))
        copies.append(pltpu.async_copy(ief_hbm.at[fi0.at[sl]], i0.at[sl], sem))
        copies.append(pltpu.async_copy(ief_hbm.at[fi1.at[sl]], i1.at[sl], sem))
    for c in copies:
        c.wait()

    wv = w_v[...]
    w0 = wv[0]
    w1 = wv[1]
    bias = wv[2]

    def step(k, _):
        sl = pl.ds(k * _LANES, _LANES)
        ru = idx_u[sl]
        ri = idx_i[sl]
        in_u = ru < _MAIN
        in_i = ri < _MAIN
        tu = jnp.where(in_u, _MAIN, ru) - _MAIN
        ti = jnp.where(in_i, _MAIN, ri) - _MAIN
        tu0 = plsc.load_gather(tailu_v, [tu])
        tu1 = plsc.load_gather(tailu_v, [tu + _NTAIL])
        ti0 = plsc.load_gather(taili_v, [ti])
        ti1 = plsc.load_gather(taili_v, [ti + _NTAIL])
        u0v = jnp.where(in_u, u0[sl], tu0)
        u1v = jnp.where(in_u, u1[sl], tu1)
        i0v = jnp.where(in_i, i0[sl], ti0)
        i1v = jnp.where(in_i, i1[sl], ti1)
        out_v[sl] = (u0v + i0v) * w0 + (u1v + i1v) * w1 + bias
        return 0

    lax.fori_loop(0, _BPW // _LANES, step, 0, unroll=2)

    pltpu.sync_copy(out_v, out_hbm.at[pl.ds(base, _BPW)])


_sc_kernel = functools.partial(
    pl.kernel,
    out_type=jax.ShapeDtypeStruct((_BATCH,), jnp.float32),
    mesh=plsc.VectorSubcoreMesh(core_axis_name="c", subcore_axis_name="s",
                                num_cores=_NC, num_subcores=_NS),
    scratch_types=[
        pltpu.VMEM((_BPW,), jnp.int32),
        pltpu.VMEM((_BPW,), jnp.int32),
        pltpu.VMEM((_BPW,), jnp.int32),
        pltpu.VMEM((_BPW,), jnp.int32),
        pltpu.VMEM((_BPW,), jnp.int32),
        pltpu.VMEM((_BPW,), jnp.int32),
        pltpu.VMEM((_BPW,), jnp.float32),
        pltpu.VMEM((_BPW,), jnp.float32),
        pltpu.VMEM((_BPW,), jnp.float32),
        pltpu.VMEM((_BPW,), jnp.float32),
        pltpu.VMEM((2 * _NTAIL,), jnp.float32),
        pltpu.VMEM((2 * _NTAIL,), jnp.float32),
        pltpu.VMEM((_BPW,), jnp.float32),
        pltpu.VMEM((_LANES,), jnp.float32),
        pltpu.SemaphoreType.DMA,
    ],
    compiler_params=pltpu.CompilerParams(use_tc_tiling_on_sc=False,
                                         needs_layout_passes=False),
)(_sc_body)

_B = _MAIN // 128


def _native_view(table):
    m = table[:_MAIN]
    y = with_layout_constraint(
        m.reshape(_B, 128, _DIM),
        Layout(major_to_minor=(0, 2, 1), tiling=((2, 128),)))
    z = with_layout_constraint(
        y.transpose(0, 2, 1),
        Layout(major_to_minor=(0, 1, 2), tiling=((2, 128),)))
    flat = z.reshape(_B * 128 * _DIM)
    tail = jnp.ravel(table[_MAIN:].T)
    return flat, tail


def kernel(users, items, UE, IE, FC_W, FC_b):
    wb = jnp.concatenate(
        [FC_W.reshape(_DIM), FC_b, jnp.zeros((_LANES - _DIM - 1,), jnp.float32)])
    uef, tailu = _native_view(UE)
    ief, taili = _native_view(IE)
    out = _sc_kernel(users, items, uef, ief, tailu, taili, wb)
    return out.reshape(_BATCH, 1)


# speedup vs baseline: 65.2960x; 1.0611x over previous
import functools

import jax
import jax.numpy as jnp
from jax import lax
from jax.experimental import pallas as pl
from jax.experimental.layout import Layout, with_layout_constraint
from jax.experimental.pallas import tpu as pltpu
from jax.experimental.pallas import tpu_sc as plsc

_BATCH = 16384
_DIM = 2
_NROWS = 1000000
_MAIN = 999936
_NTAIL = _NROWS - _MAIN
_NC = 2
_NS = 16
_LANES = 16
_NW = _NC * _NS
_BPW = _BATCH // _NW
_CHUNK = 128
_NCHUNK = _BPW // _CHUNK
_TU = _LANES
_TI = _LANES + 2 * _NTAIL
_SIDE = _LANES + 4 * _NTAIL


def _sc_body(users_hbm, items_hbm, uef_hbm, ief_hbm, side_hbm, out_hbm,
             idx_u, idx_i, fu0, fu1, fi0, fi1, u0, u1, i0, i1,
             side_v, out_v, sem):
    wid = lax.axis_index("s") * _NC + lax.axis_index("c")
    base = wid * _BPW

    pltpu.sync_copy(users_hbm.at[pl.ds(base, _BPW)], idx_u)
    pltpu.sync_copy(items_hbm.at[pl.ds(base, _BPW)], idx_i)
    pltpu.sync_copy(side_hbm, side_v)

    copies = []
    for j in range(_NCHUNK):
        csl = pl.ds(j * _CHUNK, _CHUNK)

        def mkidx(k, _, j=j):
            sl = pl.ds(j * _CHUNK + k * _LANES, _LANES)
            ru = idx_u[sl]
            ri = idx_i[sl]
            rum = jnp.where(ru < _MAIN, ru, 0)
            rim = jnp.where(ri < _MAIN, ri, 0)
            bu = lax.shift_right_logical(rum, 7) * 256 + (rum & 127)
            bi = lax.shift_right_logical(rim, 7) * 256 + (rim & 127)
            fu0[sl] = bu
            fu1[sl] = bu + 128
            fi0[sl] = bi
            fi1[sl] = bi + 128
            return 0

        lax.fori_loop(0, _CHUNK // _LANES, mkidx, 0, unroll=4)
        copies.append(pltpu.async_copy(uef_hbm.at[fu0.at[csl]], u0.at[csl], sem))
        copies.append(pltpu.async_copy(uef_hbm.at[fu1.at[csl]], u1.at[csl], sem))
        copies.append(pltpu.async_copy(ief_hbm.at[fi0.at[csl]], i0.at[csl], sem))
        copies.append(pltpu.async_copy(ief_hbm.at[fi1.at[csl]], i1.at[csl], sem))
    for c in copies:
        c.wait()

    wv = side_v[pl.ds(0, _LANES)]
    w0 = wv[0]
    w1 = wv[1]
    bias = wv[2]

    def step(k, _):
        sl = pl.ds(k * _LANES, _LANES)
        ru = idx_u[sl]
        ri = idx_i[sl]
        in_u = ru < _MAIN
        in_i = ri < _MAIN
        tu = jnp.where(in_u, _MAIN, ru) - _MAIN
        ti = jnp.where(in_i, _MAIN, ri) - _MAIN
        tu0 = plsc.load_gather(side_v, [tu + _TU])
        tu1 = plsc.load_gather(side_v, [tu + (_TU + _NTAIL)])
        ti0 = plsc.load_gather(side_v, [ti + _TI])
        ti1 = plsc.load_gather(side_v, [ti + (_TI + _NTAIL)])
        u0v = jnp.where(in_u, u0[sl], tu0)
        u1v = jnp.where(in_u, u1[sl], tu1)
        i0v = jnp.where(in_i, i0[sl], ti0)
        i1v = jnp.where(in_i, i1[sl], ti1)
        out_v[sl] = (u0v + i0v) * w0 + (u1v + i1v) * w1 + bias
        return 0

    lax.fori_loop(0, _BPW // _LANES, step, 0, unroll=2)

    pltpu.sync_copy(out_v, out_hbm.at[pl.ds(base, _BPW)])


_sc_kernel = functools.partial(
    pl.kernel,
    out_type=jax.ShapeDtypeStruct((_BATCH,), jnp.float32),
    mesh=plsc.VectorSubcoreMesh(core_axis_name="c", subcore_axis_name="s",
                                num_cores=_NC, num_subcores=_NS),
    scratch_types=[
        pltpu.VMEM((_BPW,), jnp.int32),
        pltpu.VMEM((_BPW,), jnp.int32),
        pltpu.VMEM((_BPW,), jnp.int32),
        pltpu.VMEM((_BPW,), jnp.int32),
        pltpu.VMEM((_BPW,), jnp.int32),
        pltpu.VMEM((_BPW,), jnp.int32),
        pltpu.VMEM((_BPW,), jnp.float32),
        pltpu.VMEM((_BPW,), jnp.float32),
        pltpu.VMEM((_BPW,), jnp.float32),
        pltpu.VMEM((_BPW,), jnp.float32),
        pltpu.VMEM((_SIDE,), jnp.float32),
        pltpu.VMEM((_BPW,), jnp.float32),
        pltpu.SemaphoreType.DMA,
    ],
    compiler_params=pltpu.CompilerParams(use_tc_tiling_on_sc=False,
                                         needs_layout_passes=False),
)(_sc_body)

_B = _MAIN // 128


def _native_view(table):
    m = table[:_MAIN]
    y = with_layout_constraint(
        m.reshape(_B, 128, _DIM),
        Layout(major_to_minor=(0, 2, 1), tiling=((2, 128),)))
    z = with_layout_constraint(
        y.transpose(0, 2, 1),
        Layout(major_to_minor=(0, 1, 2), tiling=((2, 128),)))
    flat = z.reshape(_B * 128 * _DIM)
    tail = jnp.ravel(table[_MAIN:].T)
    return flat, tail


def kernel(users, items, UE, IE, FC_W, FC_b):
    uef, tailu = _native_view(UE)
    ief, taili = _native_view(IE)
    side = jnp.concatenate(
        [FC_W.reshape(_DIM), FC_b,
         jnp.zeros((_LANES - _DIM - 1,), jnp.float32), tailu, taili])
    out = _sc_kernel(users, items, uef, ief, side)
    return out.reshape(_BATCH, 1)


# speedup vs baseline: 66.7403x; 1.0221x over previous
import functools

import jax
import jax.numpy as jnp
from jax import lax
from jax.experimental import pallas as pl
from jax.experimental.layout import Layout, with_layout_constraint
from jax.experimental.pallas import tpu as pltpu
from jax.experimental.pallas import tpu_sc as plsc

_BATCH = 16384
_DIM = 2
_NROWS = 1000000
_MAIN = 999936
_NTAIL = _NROWS - _MAIN
_NC = 2
_NS = 16
_LANES = 16
_NW = _NC * _NS
_BPW = _BATCH // _NW
_CHUNK = 256
_NCHUNK = _BPW // _CHUNK
_TU = _LANES
_TI = _LANES + 2 * _NTAIL
_SIDE = _LANES + 4 * _NTAIL


def _sc_body(users_hbm, items_hbm, uef_hbm, ief_hbm, side_hbm, out_hbm,
             idx_u, idx_i, fu0, fu1, fi0, fi1, u0, u1, i0, i1,
             side_v, out_v, sem):
    wid = lax.axis_index("s") * _NC + lax.axis_index("c")
    base = wid * _BPW

    stage = [
        pltpu.async_copy(users_hbm.at[pl.ds(base, _BPW)], idx_u, sem),
        pltpu.async_copy(items_hbm.at[pl.ds(base, _BPW)], idx_i, sem),
        pltpu.async_copy(side_hbm, side_v, sem),
    ]
    for c in stage:
        c.wait()

    copies = []
    for j in range(_NCHUNK):
        csl = pl.ds(j * _CHUNK, _CHUNK)

        def mkidx(k, _, j=j):
            sl = pl.ds(j * _CHUNK + k * _LANES, _LANES)
            ru = idx_u[sl]
            ri = idx_i[sl]
            rum = jnp.where(ru < _MAIN, ru, 0)
            rim = jnp.where(ri < _MAIN, ri, 0)
            bu = lax.shift_right_logical(rum, 7) * 256 + (rum & 127)
            bi = lax.shift_right_logical(rim, 7) * 256 + (rim & 127)
            fu0[sl] = bu
            fu1[sl] = bu + 128
            fi0[sl] = bi
            fi1[sl] = bi + 128
            return 0

        lax.fori_loop(0, _CHUNK // _LANES, mkidx, 0, unroll=4)
        copies.append(pltpu.async_copy(uef_hbm.at[fu0.at[csl]], u0.at[csl], sem))
        copies.append(pltpu.async_copy(uef_hbm.at[fu1.at[csl]], u1.at[csl], sem))
        copies.append(pltpu.async_copy(ief_hbm.at[fi0.at[csl]], i0.at[csl], sem))
        copies.append(pltpu.async_copy(ief_hbm.at[fi1.at[csl]], i1.at[csl], sem))
    for c in copies:
        c.wait()

    wv = side_v[pl.ds(0, _LANES)]
    w0 = wv[0]
    w1 = wv[1]
    bias = wv[2]

    def step(k, _):
        sl = pl.ds(k * _LANES, _LANES)
        ru = idx_u[sl]
        ri = idx_i[sl]
        in_u = ru < _MAIN
        in_i = ri < _MAIN
        tu = jnp.where(in_u, _MAIN, ru) - _MAIN
        ti = jnp.where(in_i, _MAIN, ri) - _MAIN
        tu0 = plsc.load_gather(side_v, [tu + _TU])
        tu1 = plsc.load_gather(side_v, [tu + (_TU + _NTAIL)])
        ti0 = plsc.load_gather(side_v, [ti + _TI])
        ti1 = plsc.load_gather(side_v, [ti + (_TI + _NTAIL)])
        u0v = jnp.where(in_u, u0[sl], tu0)
        u1v = jnp.where(in_u, u1[sl], tu1)
        i0v = jnp.where(in_i, i0[sl], ti0)
        i1v = jnp.where(in_i, i1[sl], ti1)
        out_v[sl] = (u0v + i0v) * w0 + (u1v + i1v) * w1 + bias
        return 0

    lax.fori_loop(0, _BPW // _LANES, step, 0, unroll=2)

    pltpu.sync_copy(out_v, out_hbm.at[pl.ds(base, _BPW)])


_sc_kernel = functools.partial(
    pl.kernel,
    out_type=jax.ShapeDtypeStruct((_BATCH,), jnp.float32),
    mesh=plsc.VectorSubcoreMesh(core_axis_name="c", subcore_axis_name="s",
                                num_cores=_NC, num_subcores=_NS),
    scratch_types=[
        pltpu.VMEM((_BPW,), jnp.int32),
        pltpu.VMEM((_BPW,), jnp.int32),
        pltpu.VMEM((_BPW,), jnp.int32),
        pltpu.VMEM((_BPW,), jnp.int32),
        pltpu.VMEM((_BPW,), jnp.int32),
        pltpu.VMEM((_BPW,), jnp.int32),
        pltpu.VMEM((_BPW,), jnp.float32),
        pltpu.VMEM((_BPW,), jnp.float32),
        pltpu.VMEM((_BPW,), jnp.float32),
        pltpu.VMEM((_BPW,), jnp.float32),
        pltpu.VMEM((_SIDE,), jnp.float32),
        pltpu.VMEM((_BPW,), jnp.float32),
        pltpu.SemaphoreType.DMA,
    ],
    compiler_params=pltpu.CompilerParams(use_tc_tiling_on_sc=False,
                                         needs_layout_passes=False),
)(_sc_body)

_B = _MAIN // 128


def _native_view(table):
    m = table[:_MAIN]
    y = with_layout_constraint(
        m.reshape(_B, 128, _DIM),
        Layout(major_to_minor=(0, 2, 1), tiling=((2, 128),)))
    z = with_layout_constraint(
        y.transpose(0, 2, 1),
        Layout(major_to_minor=(0, 1, 2), tiling=((2, 128),)))
    flat = z.reshape(_B * 128 * _DIM)
    tail = jnp.ravel(table[_MAIN:].T)
    return flat, tail


def kernel(users, items, UE, IE, FC_W, FC_b):
    uef, tailu = _native_view(UE)
    ief, taili = _native_view(IE)
    side = jnp.concatenate(
        [FC_W.reshape(_DIM), FC_b,
         jnp.zeros((_LANES - _DIM - 1,), jnp.float32), tailu, taili])
    out = _sc_kernel(users, items, uef, ief, side)
    return out.reshape(_BATCH, 1)
